# baseline (speedup 1.0000x reference)
import functools

import jax
import jax.numpy as jnp
from jax import lax
from jax.experimental import pallas as pl
from jax.experimental.pallas import tpu as pltpu
from jax.experimental.pallas import tpu_sc as plsc

ALPHA = 0.2
NEG = -1e30


def _dense_body(x_ref, w_ref, b_ref, wk_ref, bk_ref, wq_ref, bq_ref,
                la_ref, h_ref, key_ref, q_ref, sala_ref):
    x = x_ref[...]
    h = jnp.dot(x, w_ref[...], preferred_element_type=jnp.float32) + b_ref[...]
    h_ref[...] = h
    key_ref[...] = jnp.dot(x, wk_ref[...], preferred_element_type=jnp.float32) + bk_ref[...]
    q_ref[...] = jnp.dot(x, wq_ref[...], preferred_element_type=jnp.float32) + bq_ref[...]
    sala_ref[...] = jnp.dot(h, la_ref[...], preferred_element_type=jnp.float32)


def _dense(x, W, B, Wk, Bk, Wq, Bq, la):
    n, fin = x.shape
    fout = W.shape[1]
    ah = Wk.shape[1]
    blk = 1280
    grid = (n // blk,)
    out_shapes = (
        jax.ShapeDtypeStruct((n, fout), jnp.float32),
        jax.ShapeDtypeStruct((n, ah), jnp.float32),
        jax.ShapeDtypeStruct((n, ah), jnp.float32),
        jax.ShapeDtypeStruct((n, 1), jnp.float32),
    )
    row_spec = lambda w: pl.BlockSpec((blk, w), lambda i: (i, 0))
    full = lambda a: pl.BlockSpec(a.shape, lambda i: (0,) * a.ndim)
    return pl.pallas_call(
        _dense_body,
        grid=grid,
        in_specs=[row_spec(fin), full(W), full(B), full(Wk), full(Bk),
                  full(Wq), full(Bq), full(la)],
        out_specs=(row_spec(fout), row_spec(ah), row_spec(ah), row_spec(1)),
        out_shape=out_shapes,
    )(x, W, B, Wk, Bk, Wq, Bq, la)


R = 32
CH = 32
L = 16


def _splat_i32(v):
    return jnp.full((L,), v, dtype=jnp.int32)


def _merge16(rk, rv, ck, cv):
    rb_k = lax.rev(ck, (0,))
    rb_v = lax.rev(cv, (0,))
    take = rk >= rb_k
    tk = jnp.where(take, rk, rb_k)
    tv = jnp.where(take, rv, rb_v)
    return plsc.sort_key_val(tk, tv, descending=True)


def _sc_body(rf_hbm, adj_hbm, key_hbm, q_hbm, nh_hbm, sala_hbm,
             fh_hbm, ex_hbm,
             rfb, keybuf, ownbuf, qchunk, achunk, nhsub, fhbuf, ebuf,
             sala_v, sem_q, sem_nh, sem_adj):
    nc = 2
    wid = lax.axis_index("s") * nc + lax.axis_index("c")
    nchunks = 10

    pltpu.sync_copy(sala_hbm, sala_v)

    iota = lax.iota(jnp.int32, L)

    def chunk_body(c, _):
        cid = wid * nchunks + c
        g0 = cid * CH
        pltpu.sync_copy(rf_hbm.at[pl.ds(8 * cid, 8)], rfb)
        pltpu.sync_copy(key_hbm.at[pl.ds(g0, CH)], keybuf)
        pltpu.sync_copy(nh_hbm.at[pl.ds(g0, CH)], ownbuf)

        cps = []
        for rr in range(8):
            cps.append(pltpu.async_copy(
                q_hbm.at[rfb.at[rr]], qchunk.at[pl.ds(128 * rr, 128)],
                sem_q))
            cps.append(pltpu.async_copy(
                adj_hbm.at[rfb.at[rr]], achunk.at[pl.ds(128 * rr, 128)],
                sem_adj))
        for cp in cps:
            cp.wait()

        def node_body(li, lis, s):
            kv = keybuf[li, :]
            qb = 32 * li
            att0 = jnp.zeros((L,), jnp.float32)
            att1 = jnp.zeros((L,), jnp.float32)
            for j in range(R):
                sj = jnp.sum(kv * qchunk[qb + j, :])
                if j < L:
                    att0 = jnp.where(iota == j, sj, att0)
                else:
                    att1 = jnp.where(iota == (j - L), sj, att1)

            att0 = jnp.where(att0 >= 0, att0, ALPHA * att0)
            att1 = jnp.where(att1 >= 0, att1, ALPHA * att1)
            rfv0 = rfb[2 * s + lis // 4, pl.ds((lis % 4) * R, L)]
            rfv1 = rfb[2 * s + lis // 4, pl.ds((lis % 4) * R + L, L)]
            att0 = jnp.where(rfv0 != 9999, att0, NEG)
            att1 = jnp.where(rfv1 != 9999, att1, NEG)

            for _ in range(2):
                m = jnp.max(jnp.maximum(att0, att1))
                e0 = jnp.exp(att0 - m)
                e1 = jnp.exp(att1 - m)
                ssum = jnp.sum(e0 + e1)
                att0 = e0 / ssum
                att1 = e1 / ssum
            nb = 32 * lis
            acc = [jnp.zeros((L,), jnp.float32) for _ in range(8)]
            for j in range(R):
                src = att0 if j < L else att1
                wj = jnp.sum(jnp.where(iota == (j % L), src, 0.0))
                for f in range(8):
                    acc[f] = acc[f] + wj * nhsub[nb + j, pl.ds(f * L, L)]
            for f in range(8):
                o = acc[f] + ownbuf[li, pl.ds(f * L, L)]
                fhbuf[li, pl.ds(f * L, L)] = jnp.maximum(o, 0.0)

            hi_k = jnp.full((L,), -3.4e38, jnp.float32)
            hi_v = jnp.zeros((L,), jnp.int32)
            lo_k = jnp.full((L,), -3.4e38, jnp.float32)
            lo_v = jnp.zeros((L,), jnp.int32)
            ab = 32 * li

            def topk_body(r, carry):
                hi_k, hi_v, lo_k, lo_v = carry
                for half in range(2):
                    nvec = achunk[ab + r, pl.ds(half * L, L)]
                    kvec = plsc.load_gather(sala_v, [nvec])
                    ck, cv = plsc.sort_key_val(kvec, nvec, descending=True)
                    tk, tv = _merge16(lo_k, lo_v, ck, cv)
                    rt_k = lax.rev(tk, (0,))
                    rt_v = lax.rev(tv, (0,))
                    take = hi_k >= rt_k
                    nh_k = jnp.where(take, hi_k, rt_k)
                    nh_v = jnp.where(take, hi_v, rt_v)
                    nl_k = jnp.where(take, rt_k, hi_k)
                    nl_v = jnp.where(take, rt_v, hi_v)
                    hi_k, hi_v = plsc.sort_key_val(nh_k, nh_v, descending=True)
                    lo_k, lo_v = plsc.sort_key_val(nl_k, nl_v, descending=True)
                return hi_k, hi_v, lo_k, lo_v

            hi_k, hi_v, lo_k, lo_v = lax.fori_loop(
                0, R, topk_body, (hi_k, hi_v, lo_k, lo_v))
            ebuf[li, pl.ds(0, L)] = hi_v
            ebuf[li, pl.ds(L, L)] = lo_v

        for s in range(4):
            cp1 = pltpu.async_copy(
                nh_hbm.at[rfb.at[2 * s]], nhsub.at[pl.ds(0, 128)], sem_nh)
            cp2 = pltpu.async_copy(
                nh_hbm.at[rfb.at[2 * s + 1]], nhsub.at[pl.ds(128, 128)],
                sem_nh)
            cp1.wait()
            cp2.wait()

            def sub_body(lis, _, s=s):
                node_body(8 * s + lis, lis, s)
                return 0

            lax.fori_loop(0, 8, sub_body, 0)

        pltpu.sync_copy(fhbuf, fh_hbm.at[pl.ds(g0, CH)])
        pltpu.sync_copy(ebuf, ex_hbm.at[pl.ds(g0, CH)])
        return 0

    lax.fori_loop(0, nchunks, chunk_body, 0)


def _sc_call(rf_pad, adj, key_pad, q_pad, nh_pad, sala_lut):
    npad = nh_pad.shape[0]
    mesh = plsc.VectorSubcoreMesh(core_axis_name="c", subcore_axis_name="s")
    fn = pl.kernel(
        _sc_body,
        mesh=mesh,
        compiler_params=pltpu.CompilerParams(
            needs_layout_passes=False, use_tc_tiling_on_sc=False),
        out_type=(
            jax.ShapeDtypeStruct((npad, 128), jnp.float32),
            jax.ShapeDtypeStruct((npad, R), jnp.int32),
        ),
        scratch_types=[
            pltpu.VMEM((8, 128), jnp.int32),
            pltpu.VMEM((CH, 16), jnp.float32),
            pltpu.VMEM((CH, 128), jnp.float32),
            pltpu.VMEM((CH * R, 16), jnp.float32),
            pltpu.VMEM((CH * R, R), jnp.int32),
            pltpu.VMEM((256, 128), jnp.float32),
            pltpu.VMEM((CH, 128), jnp.float32),
            pltpu.VMEM((CH, R), jnp.int32),
            pltpu.VMEM((10000,), jnp.float32),
            pltpu.SemaphoreType.DMA,
            pltpu.SemaphoreType.DMA,
            pltpu.SemaphoreType.DMA,
        ],
    )
    return fn(rf_pad, adj, key_pad, q_pad, nh_pad, sala_lut)


def kernel(input, receptive_field, adj, la_simple, ra_simple, Bla_simple,
           Bra_simple, W, B, Wk, Bk, Wq, Bq):
    x = input
    n = x.shape[0]
    r = receptive_field.shape[2]
    npad = 10240
    rf1 = receptive_field[0]

    xpad = jnp.pad(x, ((0, npad - n), (0, 0)))
    rf_pad = jnp.pad(rf1, ((0, npad - n), (0, 0))).reshape(npad // 4, 128)

    nh_pad, key_pad, q_pad, sala_pad = _dense(
        xpad, W[0], B[0], Wk, Bk, Wq, Bq, la_simple)
    sala_lut = sala_pad[:n, 0].at[n - 1].set(NEG)

    fh_pad, ex_pad = _sc_call(rf_pad, adj, key_pad, q_pad, nh_pad, sala_lut)

    final_h = fh_pad[:n]
    expand = ex_pad[:n]
    rf_out = jnp.concatenate([receptive_field, expand[None]], axis=0)
    return final_h, rf_out


# speedup vs baseline: 1.3318x; 1.3318x over previous
import functools

import jax
import jax.numpy as jnp
from jax import lax
from jax.experimental import pallas as pl
from jax.experimental.pallas import tpu as pltpu
from jax.experimental.pallas import tpu_sc as plsc

ALPHA = 0.2
NEG = -1e30


def _dense_body(x_ref, w_ref, b_ref, wk_ref, bk_ref, wq_ref, bq_ref,
                la_ref, h_ref, key_ref, q_ref, sala_ref):
    x = x_ref[...]
    h = jnp.dot(x, w_ref[...], preferred_element_type=jnp.float32) + b_ref[...]
    h_ref[...] = h
    key_ref[...] = jnp.dot(x, wk_ref[...], preferred_element_type=jnp.float32) + bk_ref[...]
    q_ref[...] = jnp.dot(x, wq_ref[...], preferred_element_type=jnp.float32) + bq_ref[...]
    sala_ref[...] = jnp.dot(h, la_ref[...], preferred_element_type=jnp.float32)


def _dense(x, W, B, Wk, Bk, Wq, Bq, la):
    n, fin = x.shape
    fout = W.shape[1]
    ah = Wk.shape[1]
    blk = 1280
    grid = (n // blk,)
    out_shapes = (
        jax.ShapeDtypeStruct((n, fout), jnp.float32),
        jax.ShapeDtypeStruct((n, ah), jnp.float32),
        jax.ShapeDtypeStruct((n, ah), jnp.float32),
        jax.ShapeDtypeStruct((n, 1), jnp.float32),
    )
    row_spec = lambda w: pl.BlockSpec((blk, w), lambda i: (i, 0))
    full = lambda a: pl.BlockSpec(a.shape, lambda i: (0,) * a.ndim)
    return pl.pallas_call(
        _dense_body,
        grid=grid,
        in_specs=[row_spec(fin), full(W), full(B), full(Wk), full(Bk),
                  full(Wq), full(Bq), full(la)],
        out_specs=(row_spec(fout), row_spec(ah), row_spec(ah), row_spec(1)),
        out_shape=out_shapes,
    )(x, W, B, Wk, Bk, Wq, Bq, la)


R = 32
CH = 32
L = 16


def _splat_i32(v):
    return jnp.full((L,), v, dtype=jnp.int32)


def _merge16(rk, rv, ck, cv):
    rb_k = lax.rev(ck, (0,))
    rb_v = lax.rev(cv, (0,))
    take = rk >= rb_k
    tk = jnp.where(take, rk, rb_k)
    tv = jnp.where(take, rv, rb_v)
    return plsc.sort_key_val(tk, tv, descending=True)


def _sc_body(rf_hbm, adj_hbm, key_hbm, q_hbm, nh_hbm, sala_hbm,
             fh_hbm, ex_hbm,
             rfb, keybuf, ownbuf, qchunk, achunk, nhsub, fhbuf, ebuf,
             sala_v, sem_q, sem_nh, sem_adj):
    nc = 2
    wid = lax.axis_index("s") * nc + lax.axis_index("c")
    nchunks = 10

    pltpu.sync_copy(sala_hbm, sala_v)

    iota = lax.iota(jnp.int32, L)

    def chunk_body(c, _):
        cid = wid * nchunks + c
        g0 = cid * CH
        pltpu.sync_copy(rf_hbm.at[pl.ds(8 * cid, 8)], rfb)
        pltpu.sync_copy(key_hbm.at[pl.ds(g0, CH)], keybuf)
        pltpu.sync_copy(nh_hbm.at[pl.ds(g0, CH)], ownbuf)

        cps = []
        for rr in range(8):
            cps.append(pltpu.async_copy(
                q_hbm.at[rfb.at[rr]], qchunk.at[pl.ds(128 * rr, 128)],
                sem_q))
            cps.append(pltpu.async_copy(
                adj_hbm.at[rfb.at[rr]], achunk.at[pl.ds(128 * rr, 128)],
                sem_adj))
        for cp in cps:
            cp.wait()

        def node_body(li, lis, s):
            kv = keybuf[li, :]
            qb = 32 * li
            att0 = jnp.zeros((L,), jnp.float32)
            att1 = jnp.zeros((L,), jnp.float32)
            for j in range(R):
                sj = jnp.sum(kv * qchunk[qb + j, :])
                if j < L:
                    att0 = jnp.where(iota == j, sj, att0)
                else:
                    att1 = jnp.where(iota == (j - L), sj, att1)

            att0 = jnp.where(att0 >= 0, att0, ALPHA * att0)
            att1 = jnp.where(att1 >= 0, att1, ALPHA * att1)
            rfv0 = rfb[2 * s + lis // 4, pl.ds((lis % 4) * R, L)]
            rfv1 = rfb[2 * s + lis // 4, pl.ds((lis % 4) * R + L, L)]
            att0 = jnp.where(rfv0 != 9999, att0, NEG)
            att1 = jnp.where(rfv1 != 9999, att1, NEG)

            for _ in range(2):
                m = jnp.max(jnp.maximum(att0, att1))
                e0 = jnp.exp(att0 - m)
                e1 = jnp.exp(att1 - m)
                ssum = jnp.sum(e0 + e1)
                att0 = e0 / ssum
                att1 = e1 / ssum
            nb = 32 * lis
            acc = [jnp.zeros((L,), jnp.float32) for _ in range(8)]
            for j in range(R):
                src = att0 if j < L else att1
                wj = jnp.sum(jnp.where(iota == (j % L), src, 0.0))
                for f in range(8):
                    acc[f] = acc[f] + wj * nhsub[nb + j, pl.ds(f * L, L)]
            for f in range(8):
                o = acc[f] + ownbuf[li, pl.ds(f * L, L)]
                fhbuf[li, pl.ds(f * L, L)] = jnp.maximum(o, 0.0)

            hi_k = jnp.full((L,), -3.4e38, jnp.float32)
            hi_v = jnp.zeros((L,), jnp.int32)
            lo_k = jnp.full((L,), -3.4e38, jnp.float32)
            lo_v = jnp.zeros((L,), jnp.int32)
            ab = 32 * li

            def topk_body(r, carry):
                hi_k, hi_v, lo_k, lo_v = carry
                n0 = achunk[ab + r, pl.ds(0, L)]
                n1 = achunk[ab + r, pl.ds(L, L)]
                k0 = plsc.load_gather(sala_v, [n0])
                k1 = plsc.load_gather(sala_v, [n1])
                c0k, c0v = plsc.sort_key_val(k0, n0, descending=True)
                c1k, c1v = plsc.sort_key_val(k1, n1, descending=True)
                r1k = lax.rev(c1k, (0,))
                r1v = lax.rev(c1v, (0,))
                t = c0k >= r1k
                chk = jnp.where(t, c0k, r1k)
                chv = jnp.where(t, c0v, r1v)
                clk = jnp.where(t, r1k, c0k)
                clv = jnp.where(t, r1v, c0v)
                chk, chv = plsc.sort_key_val(chk, chv, descending=True)
                clk, clv = plsc.sort_key_val(clk, clv, descending=True)
                rlk = lax.rev(clk, (0,))
                rlv = lax.rev(clv, (0,))
                rhk = lax.rev(chk, (0,))
                rhv = lax.rev(chv, (0,))
                t1 = hi_k >= rlk
                th_k = jnp.where(t1, hi_k, rlk)
                th_v = jnp.where(t1, hi_v, rlv)
                t2 = lo_k >= rhk
                tl_k = jnp.where(t2, lo_k, rhk)
                tl_v = jnp.where(t2, lo_v, rhv)
                t3 = th_k >= tl_k
                uh_k = jnp.where(t3, th_k, tl_k)
                uh_v = jnp.where(t3, th_v, tl_v)
                ul_k = jnp.where(t3, tl_k, th_k)
                ul_v = jnp.where(t3, tl_v, th_v)
                hi_k, hi_v = plsc.sort_key_val(uh_k, uh_v, descending=True)
                lo_k, lo_v = plsc.sort_key_val(ul_k, ul_v, descending=True)
                return hi_k, hi_v, lo_k, lo_v

            hi_k, hi_v, lo_k, lo_v = lax.fori_loop(
                0, R, topk_body, (hi_k, hi_v, lo_k, lo_v))
            ebuf[li, pl.ds(0, L)] = hi_v
            ebuf[li, pl.ds(L, L)] = lo_v

        for s in range(4):
            cp1 = pltpu.async_copy(
                nh_hbm.at[rfb.at[2 * s]], nhsub.at[pl.ds(0, 128)], sem_nh)
            cp2 = pltpu.async_copy(
                nh_hbm.at[rfb.at[2 * s + 1]], nhsub.at[pl.ds(128, 128)],
                sem_nh)
            cp1.wait()
            cp2.wait()

            def sub_body(lis, _, s=s):
                node_body(8 * s + lis, lis, s)
                return 0

            lax.fori_loop(0, 8, sub_body, 0)

        pltpu.sync_copy(fhbuf, fh_hbm.at[pl.ds(g0, CH)])
        pltpu.sync_copy(ebuf, ex_hbm.at[pl.ds(g0, CH)])
        return 0

    lax.fori_loop(0, nchunks, chunk_body, 0)


def _sc_call(rf_pad, adj, key_pad, q_pad, nh_pad, sala_lut):
    npad = nh_pad.shape[0]
    mesh = plsc.VectorSubcoreMesh(core_axis_name="c", subcore_axis_name="s")
    fn = pl.kernel(
        _sc_body,
        mesh=mesh,
        compiler_params=pltpu.CompilerParams(
            needs_layout_passes=False, use_tc_tiling_on_sc=False),
        out_type=(
            jax.ShapeDtypeStruct((npad, 128), jnp.float32),
            jax.ShapeDtypeStruct((npad, R), jnp.int32),
        ),
        scratch_types=[
            pltpu.VMEM((8, 128), jnp.int32),
            pltpu.VMEM((CH, 16), jnp.float32),
            pltpu.VMEM((CH, 128), jnp.float32),
            pltpu.VMEM((CH * R, 16), jnp.float32),
            pltpu.VMEM((CH * R, R), jnp.int32),
            pltpu.VMEM((256, 128), jnp.float32),
            pltpu.VMEM((CH, 128), jnp.float32),
            pltpu.VMEM((CH, R), jnp.int32),
            pltpu.VMEM((10000,), jnp.float32),
            pltpu.SemaphoreType.DMA,
            pltpu.SemaphoreType.DMA,
            pltpu.SemaphoreType.DMA,
        ],
    )
    return fn(rf_pad, adj, key_pad, q_pad, nh_pad, sala_lut)


def kernel(input, receptive_field, adj, la_simple, ra_simple, Bla_simple,
           Bra_simple, W, B, Wk, Bk, Wq, Bq):
    x = input
    n = x.shape[0]
    r = receptive_field.shape[2]
    npad = 10240
    rf1 = receptive_field[0]

    xpad = jnp.pad(x, ((0, npad - n), (0, 0)))
    rf_pad = jnp.pad(rf1, ((0, npad - n), (0, 0))).reshape(npad // 4, 128)

    nh_pad, key_pad, q_pad, sala_pad = _dense(
        xpad, W[0], B[0], Wk, Bk, Wq, Bq, la_simple)
    sala_lut = sala_pad[:n, 0].at[n - 1].set(NEG)

    fh_pad, ex_pad = _sc_call(rf_pad, adj, key_pad, q_pad, nh_pad, sala_lut)

    final_h = fh_pad[:n]
    expand = ex_pad[:n]
    rf_out = jnp.concatenate([receptive_field, expand[None]], axis=0)
    return final_h, rf_out
